# Initial kernel scaffold; baseline (speedup 1.0000x reference)
import jax
import jax.numpy as jnp
from jax.experimental import pallas as pl


def kernel(x, table, W, b, gamma, beta):
    raise NotImplementedError("write your pallas kernel here")


# speedup vs baseline: 1.3822x; 1.3822x over previous
import functools
import math

import jax
import jax.numpy as jnp
from jax import lax
from jax.experimental import pallas as pl
from jax.experimental.pallas import tpu as pltpu
from jax.experimental.pallas import tpu_sc as plsc

D_EMBED = 128
D_MODEL = 768
SEQ = 8192

_NC = 2
_NS = 16
_NW = _NC * _NS


def _sc_gather(table, idx):
    n = idx.shape[0]
    rows_per_w = n // _NW
    ch = 128
    n_ch = rows_per_w // ch
    idx3 = idx.reshape(_NW, n_ch, ch)

    mesh = plsc.VectorSubcoreMesh(core_axis_name="c", subcore_axis_name="s")

    @functools.partial(
        pl.kernel,
        out_type=jax.ShapeDtypeStruct((n, D_EMBED), jnp.float32),
        mesh=mesh,
        scratch_types=[
            pltpu.VMEM((n_ch, ch), jnp.int32),
            pltpu.VMEM((2, ch, D_EMBED), jnp.float32),
            pltpu.SemaphoreType.DMA,
            pltpu.SemaphoreType.DMA,
        ],
    )
    def gather_kernel(table_hbm, idx_hbm, out_hbm, idx_v, rows_v, sem0, sem1):
        wid = lax.axis_index("s") * _NC + lax.axis_index("c")
        base = wid * rows_per_w
        pltpu.sync_copy(idx_hbm.at[wid], idx_v)
        sems = [sem0, sem1]
        cps = [None, None]
        for c in range(n_ch):
            buf = c % 2
            cps[buf] = pltpu.async_copy(
                table_hbm.at[idx_v.at[c]], rows_v.at[buf], sems[buf]
            )
            if c > 0:
                pbuf = (c - 1) % 2
                cps[pbuf].wait()
                pltpu.sync_copy(
                    rows_v.at[pbuf], out_hbm.at[pl.ds(base + (c - 1) * ch, ch)]
                )
        lbuf = (n_ch - 1) % 2
        cps[lbuf].wait()
        pltpu.sync_copy(
            rows_v.at[lbuf], out_hbm.at[pl.ds(base + (n_ch - 1) * ch, ch)]
        )

    return gather_kernel(table, idx3)


def _tc_dense(emb, wt, b, gamma, beta):
    n = emb.shape[0]
    tile = 512
    grid = n // tile
    scale = math.sqrt(float(D_MODEL))
    ln10x4 = 4.0 * math.log(10.0)

    def body(e_ref, wt_ref, b_ref, g_ref, bt_ref, o_ref):
        i = pl.program_id(0)
        h = jnp.dot(e_ref[...], wt_ref[...], preferred_element_type=jnp.float32)
        h = (h + b_ref[...]) * scale
        pos0 = (i * tile) % SEQ
        pos = (pos0 + lax.broadcasted_iota(jnp.int32, (tile, 1), 0)).astype(
            jnp.float32
        )
        col = lax.broadcasted_iota(jnp.int32, (1, D_MODEL), 1)
        odd = col % 2
        ceven = (col - odd).astype(jnp.float32)
        freq = jnp.exp(-ceven / float(D_MODEL) * 4.0 * math.log(10.0))
        ang = pos * freq
        pe = jnp.where(odd == 1, jnp.cos(ang), jnp.sin(ang))
        h = h + pe
        m = jnp.mean(h, axis=1, keepdims=True)
        d = h - m
        v = jnp.mean(d * d, axis=1, keepdims=True)
        o_ref[...] = d * lax.rsqrt(v + 1e-5) * g_ref[...] + bt_ref[...]

    return pl.pallas_call(
        body,
        grid=(grid,),
        in_specs=[
            pl.BlockSpec((tile, D_EMBED), lambda i: (i, 0)),
            pl.BlockSpec((D_EMBED, D_MODEL), lambda i: (0, 0)),
            pl.BlockSpec((1, D_MODEL), lambda i: (0, 0)),
            pl.BlockSpec((1, D_MODEL), lambda i: (0, 0)),
            pl.BlockSpec((1, D_MODEL), lambda i: (0, 0)),
        ],
        out_specs=pl.BlockSpec((tile, D_MODEL), lambda i: (i, 0)),
        out_shape=jax.ShapeDtypeStruct((n, D_MODEL), jnp.float32),
    )(emb, wt, b, gamma, beta)


def kernel(x, table, W, b, gamma, beta):
    bsz, seq = x.shape
    idx = x.reshape(-1).astype(jnp.int32)
    emb = _sc_gather(table, idx)
    out = _tc_dense(
        emb,
        W.T,
        b.reshape(1, D_MODEL),
        gamma.reshape(1, D_MODEL),
        beta.reshape(1, D_MODEL),
    )
    return out.reshape(bsz, seq, D_MODEL)


# speedup vs baseline: 5.2078x; 3.7677x over previous
import functools
import math

import jax
import jax.numpy as jnp
from jax import lax
from jax.experimental import pallas as pl
from jax.experimental.pallas import tpu as pltpu
from jax.experimental.pallas import tpu_sc as plsc

D_EMBED = 128
D_MODEL = 768
SEQ = 8192

_NC = 2
_NS = 16
_NW = _NC * _NS


def _sc_gather(table, idx):
    n = idx.shape[0]
    rows_per_w = n // _NW
    ch = 128
    n_ch = rows_per_w // ch
    idx3 = idx.reshape(_NW, n_ch, ch)

    mesh = plsc.VectorSubcoreMesh(core_axis_name="c", subcore_axis_name="s")

    @functools.partial(
        pl.kernel,
        out_type=jax.ShapeDtypeStruct((n, D_EMBED), jnp.float32),
        mesh=mesh,
        scratch_types=[
            pltpu.VMEM((n_ch, ch), jnp.int32),
            pltpu.VMEM((2, ch, D_EMBED), jnp.float32),
            pltpu.SemaphoreType.DMA,
            pltpu.SemaphoreType.DMA,
        ],
    )
    def gather_kernel(table_hbm, idx_hbm, out_hbm, idx_v, rows_v, sem0, sem1):
        wid = lax.axis_index("s") * _NC + lax.axis_index("c")
        base = wid * rows_per_w
        pltpu.sync_copy(idx_hbm.at[wid], idx_v)
        sems = [sem0, sem1]
        cps = [None, None]
        for c in range(n_ch):
            buf = c % 2
            cps[buf] = pltpu.async_copy(
                table_hbm.at[idx_v.at[c]], rows_v.at[buf], sems[buf]
            )
            if c > 0:
                pbuf = (c - 1) % 2
                cps[pbuf].wait()
                pltpu.sync_copy(
                    rows_v.at[pbuf], out_hbm.at[pl.ds(base + (c - 1) * ch, ch)]
                )
        lbuf = (n_ch - 1) % 2
        cps[lbuf].wait()
        pltpu.sync_copy(
            rows_v.at[lbuf], out_hbm.at[pl.ds(base + (n_ch - 1) * ch, ch)]
        )

    return gather_kernel(table, idx3)


def _tc_dense(emb, wt, b, gamma, beta):
    n = emb.shape[0]
    tile = 512
    grid = n // tile
    scale = math.sqrt(float(D_MODEL))
    ln10x4 = 4.0 * math.log(10.0)

    def body(e_ref, wt_ref, b_ref, g_ref, bt_ref, o_ref, s_ref, c_ref):
        i = pl.program_id(0)
        col = lax.broadcasted_iota(jnp.int32, (1, D_MODEL), 1)
        odd = col % 2
        ceven = (col - odd).astype(jnp.float32)
        freq = jnp.exp(-ceven / float(D_MODEL) * 4.0 * math.log(10.0))

        @pl.when(i == 0)
        def _():
            r = lax.broadcasted_iota(jnp.int32, (tile, 1), 0).astype(jnp.float32)
            ang = r * freq
            s_ref[...] = jnp.sin(ang)
            c_ref[...] = jnp.cos(ang)

        h = jnp.dot(e_ref[...], wt_ref[...], preferred_element_type=jnp.float32)
        h = (h + b_ref[...]) * scale

        pos0 = jnp.float32((i * tile) % SEQ)
        ang0 = pos0 * freq
        s0 = jnp.sin(ang0)
        c0 = jnp.cos(ang0)
        is_odd = odd == 1
        sa = jnp.where(is_odd, c0, s0)
        ca = jnp.where(is_odd, -s0, c0)
        h = h + sa * c_ref[...] + ca * s_ref[...]

        m = jnp.mean(h, axis=1, keepdims=True)
        d = h - m
        v = jnp.mean(d * d, axis=1, keepdims=True)
        o_ref[...] = d * lax.rsqrt(v + 1e-5) * g_ref[...] + bt_ref[...]

    return pl.pallas_call(
        body,
        grid=(grid,),
        in_specs=[
            pl.BlockSpec((tile, D_EMBED), lambda i: (i, 0)),
            pl.BlockSpec((D_EMBED, D_MODEL), lambda i: (0, 0)),
            pl.BlockSpec((1, D_MODEL), lambda i: (0, 0)),
            pl.BlockSpec((1, D_MODEL), lambda i: (0, 0)),
            pl.BlockSpec((1, D_MODEL), lambda i: (0, 0)),
        ],
        out_specs=pl.BlockSpec((tile, D_MODEL), lambda i: (i, 0)),
        out_shape=jax.ShapeDtypeStruct((n, D_MODEL), jnp.float32),
        scratch_shapes=[
            pltpu.VMEM((tile, D_MODEL), jnp.float32),
            pltpu.VMEM((tile, D_MODEL), jnp.float32),
        ],
    )(emb, wt, b, gamma, beta)


def kernel(x, table, W, b, gamma, beta):
    bsz, seq = x.shape
    idx = x.reshape(-1).astype(jnp.int32)
    emb = _sc_gather(table, idx)
    out = _tc_dense(
        emb,
        W.T,
        b.reshape(1, D_MODEL),
        gamma.reshape(1, D_MODEL),
        beta.reshape(1, D_MODEL),
    )
    return out.reshape(bsz, seq, D_MODEL)


# speedup vs baseline: 6.1713x; 1.1850x over previous
import functools
import math

import jax
import jax.numpy as jnp
from jax import lax
from jax.experimental import pallas as pl
from jax.experimental.pallas import tpu as pltpu
from jax.experimental.pallas import tpu_sc as plsc

D_EMBED = 128
D_MODEL = 768
SEQ = 8192

_NC = 2
_NS = 16
_NW = _NC * _NS


def _sc_gather(table, idx):
    n = idx.shape[0]
    rows_per_w = n // _NW
    ch = 128
    n_ch = rows_per_w // ch
    idx3 = idx.reshape(_NW, n_ch, ch)

    mesh = plsc.VectorSubcoreMesh(core_axis_name="c", subcore_axis_name="s")

    @functools.partial(
        pl.kernel,
        out_type=jax.ShapeDtypeStruct((n, D_EMBED), jnp.float32),
        mesh=mesh,
        scratch_types=[
            pltpu.VMEM((n_ch, ch), jnp.int32),
            pltpu.VMEM((2, ch, D_EMBED), jnp.float32),
            pltpu.SemaphoreType.DMA,
            pltpu.SemaphoreType.DMA,
        ],
    )
    def gather_kernel(table_hbm, idx_hbm, out_hbm, idx_v, rows_v, sem0, sem1):
        wid = lax.axis_index("s") * _NC + lax.axis_index("c")
        base = wid * rows_per_w
        pltpu.sync_copy(idx_hbm.at[wid], idx_v)
        sems = [sem0, sem1]
        cps = [None, None]
        for c in range(n_ch):
            buf = c % 2
            cps[buf] = pltpu.async_copy(
                table_hbm.at[idx_v.at[c]], rows_v.at[buf], sems[buf]
            )
            if c > 0:
                pbuf = (c - 1) % 2
                cps[pbuf].wait()
                pltpu.sync_copy(
                    rows_v.at[pbuf], out_hbm.at[pl.ds(base + (c - 1) * ch, ch)]
                )
        lbuf = (n_ch - 1) % 2
        cps[lbuf].wait()
        pltpu.sync_copy(
            rows_v.at[lbuf], out_hbm.at[pl.ds(base + (n_ch - 1) * ch, ch)]
        )

    return gather_kernel(table, idx3)


def _tc_dense(emb, wt, b, gamma, beta):
    n = emb.shape[0]
    tile = 1024
    sub = 512
    grid = n // tile
    scale = math.sqrt(float(D_MODEL))

    def body(e_ref, wt_ref, b_ref, g_ref, bt_ref, o_ref, s_ref, c_ref):
        i = pl.program_id(0)
        col = lax.broadcasted_iota(jnp.int32, (1, D_MODEL), 1)
        odd = col % 2
        ceven = (col - odd).astype(jnp.float32)
        freq = jnp.exp(-ceven / float(D_MODEL) * 4.0 * math.log(10.0))

        @pl.when(i == 0)
        def _():
            r = lax.broadcasted_iota(jnp.int32, (sub, 1), 0).astype(jnp.float32)
            ang = r * freq
            s_ref[...] = jnp.sin(ang)
            c_ref[...] = jnp.cos(ang)

        h = jnp.dot(e_ref[...], wt_ref[...], preferred_element_type=jnp.float32)
        h = (h + b_ref[...]) * scale

        is_odd = odd == 1
        pes = []
        for k in range(tile // sub):
            pos0 = jnp.float32((i * tile + k * sub) % SEQ)
            ang0 = pos0 * freq
            s0 = jnp.sin(ang0)
            c0 = jnp.cos(ang0)
            sa = jnp.where(is_odd, c0, s0)
            ca = jnp.where(is_odd, -s0, c0)
            pes.append(sa * c_ref[...] + ca * s_ref[...])
        h = h + jnp.concatenate(pes, axis=0)

        m = jnp.mean(h, axis=1, keepdims=True)
        d = h - m
        v = jnp.mean(d * d, axis=1, keepdims=True)
        o_ref[...] = d * lax.rsqrt(v + 1e-5) * g_ref[...] + bt_ref[...]

    return pl.pallas_call(
        body,
        grid=(grid,),
        in_specs=[
            pl.BlockSpec((tile, D_EMBED), lambda i: (i, 0)),
            pl.BlockSpec((D_EMBED, D_MODEL), lambda i: (0, 0)),
            pl.BlockSpec((1, D_MODEL), lambda i: (0, 0)),
            pl.BlockSpec((1, D_MODEL), lambda i: (0, 0)),
            pl.BlockSpec((1, D_MODEL), lambda i: (0, 0)),
        ],
        out_specs=pl.BlockSpec((tile, D_MODEL), lambda i: (i, 0)),
        out_shape=jax.ShapeDtypeStruct((n, D_MODEL), jnp.float32),
        scratch_shapes=[
            pltpu.VMEM((sub, D_MODEL), jnp.float32),
            pltpu.VMEM((sub, D_MODEL), jnp.float32),
        ],
    )(emb, wt, b, gamma, beta)


def kernel(x, table, W, b, gamma, beta):
    bsz, seq = x.shape
    idx = x.reshape(-1).astype(jnp.int32)
    emb = _sc_gather(table, idx)
    out = _tc_dense(
        emb,
        W.T,
        b.reshape(1, D_MODEL),
        gamma.reshape(1, D_MODEL),
        beta.reshape(1, D_MODEL),
    )
    return out.reshape(bsz, seq, D_MODEL)


# speedup vs baseline: 6.2095x; 1.0062x over previous
import functools
import math

import jax
import jax.numpy as jnp
from jax import lax
from jax.experimental import pallas as pl
from jax.experimental.pallas import tpu as pltpu
from jax.experimental.pallas import tpu_sc as plsc

D_EMBED = 128
D_MODEL = 768
SEQ = 8192

_NC = 2
_NS = 16
_NW = _NC * _NS


def _sc_gather(table, idx):
    n = idx.shape[0]
    rows_per_w = n // _NW
    ch = 128
    n_ch = rows_per_w // ch
    idx3 = idx.reshape(_NW, n_ch, ch)

    mesh = plsc.VectorSubcoreMesh(core_axis_name="c", subcore_axis_name="s")

    @functools.partial(
        pl.kernel,
        out_type=jax.ShapeDtypeStruct((n, D_EMBED), jnp.float32),
        mesh=mesh,
        scratch_types=[
            pltpu.VMEM((n_ch, ch), jnp.int32),
            pltpu.VMEM((2, ch, D_EMBED), jnp.float32),
            pltpu.SemaphoreType.DMA,
            pltpu.SemaphoreType.DMA,
        ],
    )
    def gather_kernel(table_hbm, idx_hbm, out_hbm, idx_v, rows_v, sem0, sem1):
        wid = lax.axis_index("s") * _NC + lax.axis_index("c")
        base = wid * rows_per_w
        pltpu.sync_copy(idx_hbm.at[wid], idx_v)
        sems = [sem0, sem1]
        cps = [None, None]
        for c in range(n_ch):
            buf = c % 2
            cps[buf] = pltpu.async_copy(
                table_hbm.at[idx_v.at[c]], rows_v.at[buf], sems[buf]
            )
            if c > 0:
                pbuf = (c - 1) % 2
                cps[pbuf].wait()
                pltpu.sync_copy(
                    rows_v.at[pbuf], out_hbm.at[pl.ds(base + (c - 1) * ch, ch)]
                )
        lbuf = (n_ch - 1) % 2
        cps[lbuf].wait()
        pltpu.sync_copy(
            rows_v.at[lbuf], out_hbm.at[pl.ds(base + (n_ch - 1) * ch, ch)]
        )

    return gather_kernel(table, idx3)


def _tc_dense(emb, wt, b, gamma, beta):
    n = emb.shape[0]
    tile = 2048
    sub = 512
    grid = n // tile
    scale = math.sqrt(float(D_MODEL))

    def body(e_ref, wt_ref, b_ref, g_ref, bt_ref, o_ref, s_ref, c_ref):
        i = pl.program_id(0)
        col = lax.broadcasted_iota(jnp.int32, (1, D_MODEL), 1)
        odd = col % 2
        ceven = (col - odd).astype(jnp.float32)
        freq = jnp.exp(-ceven / float(D_MODEL) * 4.0 * math.log(10.0))

        @pl.when(i == 0)
        def _():
            r = lax.broadcasted_iota(jnp.int32, (sub, 1), 0).astype(jnp.float32)
            ang = r * freq
            s_ref[...] = jnp.sin(ang)
            c_ref[...] = jnp.cos(ang)

        h = jnp.dot(e_ref[...], wt_ref[...], preferred_element_type=jnp.float32)
        h = (h + b_ref[...]) * scale

        is_odd = odd == 1
        pes = []
        for k in range(tile // sub):
            pos0 = jnp.float32((i * tile + k * sub) % SEQ)
            ang0 = pos0 * freq
            s0 = jnp.sin(ang0)
            c0 = jnp.cos(ang0)
            sa = jnp.where(is_odd, c0, s0)
            ca = jnp.where(is_odd, -s0, c0)
            pes.append(sa * c_ref[...] + ca * s_ref[...])
        h = h + jnp.concatenate(pes, axis=0)

        m = jnp.mean(h, axis=1, keepdims=True)
        d = h - m
        v = jnp.mean(d * d, axis=1, keepdims=True)
        o_ref[...] = d * lax.rsqrt(v + 1e-5) * g_ref[...] + bt_ref[...]

    return pl.pallas_call(
        body,
        grid=(grid,),
        in_specs=[
            pl.BlockSpec((tile, D_EMBED), lambda i: (i, 0)),
            pl.BlockSpec((D_EMBED, D_MODEL), lambda i: (0, 0)),
            pl.BlockSpec((1, D_MODEL), lambda i: (0, 0)),
            pl.BlockSpec((1, D_MODEL), lambda i: (0, 0)),
            pl.BlockSpec((1, D_MODEL), lambda i: (0, 0)),
        ],
        out_specs=pl.BlockSpec((tile, D_MODEL), lambda i: (i, 0)),
        out_shape=jax.ShapeDtypeStruct((n, D_MODEL), jnp.float32),
        scratch_shapes=[
            pltpu.VMEM((sub, D_MODEL), jnp.float32),
            pltpu.VMEM((sub, D_MODEL), jnp.float32),
        ],
    )(emb, wt, b, gamma, beta)


def kernel(x, table, W, b, gamma, beta):
    bsz, seq = x.shape
    idx = x.reshape(-1).astype(jnp.int32)
    emb = _sc_gather(table, idx)
    out = _tc_dense(
        emb,
        W.T,
        b.reshape(1, D_MODEL),
        gamma.reshape(1, D_MODEL),
        beta.reshape(1, D_MODEL),
    )
    return out.reshape(bsz, seq, D_MODEL)
